# baseline (speedup 1.0000x reference)
import functools

import jax
import jax.numpy as jnp
from jax import lax
from jax.experimental import pallas as pl
from jax.experimental.pallas import tpu as pltpu
from jax.experimental.pallas import tpu_sc as plsc

N = 10000
E = 320000
F = 128
O = 64

NC, NS = 2, 16
NW = NC * NS
CHUNK = 128
CPW = 80
NCHUNKS = CPW * NW
EPAD = NCHUNKS * CHUNK
EPW = CPW * CHUNK
DUMMY = N
DEGW = 128

RPAD = 10240
RPS = RPAD // NS
BLK = 512
NBLK = RPAD // BLK

_f32 = jnp.float32
_mesh = plsc.VectorSubcoreMesh(core_axis_name="c", subcore_axis_name="s")


@functools.partial(
    pl.kernel,
    out_type=jax.ShapeDtypeStruct((NC * RPAD, DEGW), _f32),
    mesh=_mesh,
    scratch_types=[
        pltpu.VMEM((CPW, CHUNK), jnp.int32),
        pltpu.VMEM((CHUNK, DEGW), _f32),
        pltpu.VMEM_SHARED((RPAD, DEGW), _f32),
    ],
)
def _deg_kernel(dst_hbm, ones_hbm, zrow_hbm, out_hbm, dst_v, ones_v, deg_sh):
    c = lax.axis_index("c")
    s = lax.axis_index("s")
    pltpu.sync_copy(zrow_hbm, deg_sh.at[pl.ds(s * RPS, RPS)])
    pltpu.sync_copy(ones_hbm, ones_v)
    cbase = (c * NS + s) * CPW
    pltpu.sync_copy(dst_hbm.at[pl.ds(cbase, CPW)], dst_v)
    plsc.subcore_barrier()

    @pl.loop(0, CPW)
    def _(j):
        pltpu.sync_copy(ones_v, deg_sh.at[dst_v.at[j]], add=True)

    plsc.subcore_barrier()
    pltpu.sync_copy(deg_sh.at[pl.ds(s * RPS, RPS)],
                    out_hbm.at[pl.ds(c * RPAD + s * RPS, RPS)])


@functools.partial(
    pl.kernel,
    out_type=jax.ShapeDtypeStruct((NC * RPAD, F), _f32),
    mesh=_mesh,
    scratch_types=[
        pltpu.VMEM((CPW, CHUNK), jnp.int32),
        pltpu.VMEM((2, CHUNK), jnp.int32),
        pltpu.VMEM((2, CHUNK, F), _f32),
        pltpu.VMEM_SHARED((RPAD, F), _f32),
        pltpu.SemaphoreType.DMA,
        pltpu.SemaphoreType.DMA,
        pltpu.SemaphoreType.DMA,
        pltpu.SemaphoreType.DMA,
    ],
)
def _agg_kernel(v_hbm, src_hbm, dst_hbm, zblk_hbm, out_hbm,
                src_v, dst_v, rows_v, acc_sh, rs0, rs1, ds0, ds1):
    c = lax.axis_index("c")
    s = lax.axis_index("s")
    pltpu.sync_copy(zblk_hbm, acc_sh.at[pl.ds(s * RPS, RPS)])
    cbase = (c * NS + s) * CPW
    pltpu.sync_copy(src_hbm.at[pl.ds(cbase, CPW)], src_v)
    pltpu.sync_copy(dst_hbm.at[cbase], dst_v.at[0])
    plsc.subcore_barrier()
    rsems = (rs0, rs1)
    dsems = (ds0, ds1)

    pltpu.async_copy(v_hbm.at[src_v.at[0]], rows_v.at[0], rs0)

    @pl.loop(0, CPW, step=2)
    def _(j):
        for b in range(2):
            nxt = j + b + 1

            @pl.when(nxt < CPW)
            def _():
                pltpu.async_copy(dst_hbm.at[cbase + nxt], dst_v.at[1 - b],
                                 dsems[1 - b])
                pltpu.async_copy(v_hbm.at[src_v.at[nxt]], rows_v.at[1 - b],
                                 rsems[1 - b])

            @pl.when(j + b > 0)
            def _():
                pltpu.make_async_copy(dst_hbm.at[cbase], dst_v.at[b],
                                      dsems[b]).wait()

            pltpu.make_async_copy(v_hbm.at[pl.ds(0, CHUNK)], rows_v.at[b],
                                  rs0 if b == 0 else rs1).wait()
            pltpu.sync_copy(rows_v.at[b], acc_sh.at[dst_v.at[b]], add=True)

    plsc.subcore_barrier()
    pltpu.sync_copy(acc_sh.at[pl.ds(s * RPS, RPS)],
                    out_hbm.at[pl.ds(c * RPAD + s * RPS, RPS)])


def _dinv_block(degT_blk):
    deg = degT_blk[:, 0:1] + degT_blk[:, 1:2] + 1.0
    return lax.rsqrt(deg)


def _prep_body(x_ref, w1_ref, degT_ref, v1_ref):
    xw = jnp.dot(x_ref[...], w1_ref[...], preferred_element_type=_f32,
                 precision=lax.Precision.HIGHEST)
    v1_ref[...] = xw * _dinv_block(degT_ref[...])


def _mid_body(s_ref, v1_ref, degT_ref, b1_ref, v2_ref):
    dinv = _dinv_block(degT_ref[...])
    pre = (s_ref[0] + s_ref[1] + v1_ref[...]) * dinv + b1_ref[...]
    v2_ref[...] = jnp.maximum(pre, 0.0) * dinv


def _out_body(t_ref, v2_ref, degT_ref, wmu_ref, bmu_ref, wls_ref, bls_ref,
              mu_ref, ls_ref):
    dinv = _dinv_block(degT_ref[...])
    q = (t_ref[0] + t_ref[1] + v2_ref[...]) * dinv
    mu_ref[...] = jnp.dot(q, wmu_ref[...], preferred_element_type=_f32,
                          precision=lax.Precision.HIGHEST) + bmu_ref[...]
    ls_ref[...] = jnp.dot(q, wls_ref[...], preferred_element_type=_f32,
                          precision=lax.Precision.HIGHEST) + bls_ref[...]


_row = lambda i: (i, 0)
_full2 = pl.BlockSpec((2, RPAD), lambda i: (0, 0))

_prep_call = pl.pallas_call(
    _prep_body,
    grid=(NBLK,),
    in_specs=[pl.BlockSpec((BLK, F), _row),
              pl.BlockSpec((F, F), lambda i: (0, 0)),
              pl.BlockSpec((BLK, 2), _row)],
    out_specs=pl.BlockSpec((BLK, F), _row),
    out_shape=jax.ShapeDtypeStruct((RPAD, F), _f32),
)

_mid_call = pl.pallas_call(
    _mid_body,
    grid=(NBLK,),
    in_specs=[pl.BlockSpec((2, BLK, F), lambda i: (0, i, 0)),
              pl.BlockSpec((BLK, F), _row),
              pl.BlockSpec((BLK, 2), _row),
              pl.BlockSpec((1, F), lambda i: (0, 0))],
    out_specs=pl.BlockSpec((BLK, F), _row),
    out_shape=jax.ShapeDtypeStruct((RPAD, F), _f32),
)

_out_call = pl.pallas_call(
    _out_body,
    grid=(NBLK,),
    in_specs=[pl.BlockSpec((2, BLK, F), lambda i: (0, i, 0)),
              pl.BlockSpec((BLK, F), _row),
              pl.BlockSpec((BLK, 2), _row),
              pl.BlockSpec((F, O), lambda i: (0, 0)),
              pl.BlockSpec((1, O), lambda i: (0, 0)),
              pl.BlockSpec((F, O), lambda i: (0, 0)),
              pl.BlockSpec((1, O), lambda i: (0, 0))],
    out_specs=[pl.BlockSpec((BLK, O), _row), pl.BlockSpec((BLK, O), _row)],
    out_shape=[jax.ShapeDtypeStruct((RPAD, O), _f32),
               jax.ShapeDtypeStruct((RPAD, O), _f32)],
)


def kernel(x, edge_index, W1, b1, Wmu, bmu, Wls, bls):
    src = edge_index[0].astype(jnp.int32)
    dst = edge_index[1].astype(jnp.int32)
    pad = EPAD - E
    fill = jnp.full((pad,), DUMMY, jnp.int32)
    src1d = jnp.concatenate([src, fill]).reshape(NCHUNKS, CHUNK)
    dst1d = jnp.concatenate([dst, fill]).reshape(NCHUNKS, CHUNK)

    ones = jnp.ones((CHUNK, DEGW), _f32)
    zrow = jnp.zeros((RPS, DEGW), _f32)
    zblk = jnp.zeros((RPS, F), _f32)

    degp = _deg_kernel(dst1d, ones, zrow)
    degT = degp.reshape(NC, RPAD, DEGW)[:, :, 0].T

    x_pad = jnp.concatenate([x, jnp.zeros((RPAD - N, F), _f32)])
    v1 = _prep_call(x_pad, W1, degT)
    s = _agg_kernel(v1, src1d, dst1d, zblk).reshape(NC, RPAD, F)
    v2 = _mid_call(s, v1, degT, b1.reshape(1, F))
    t = _agg_kernel(v2, src1d, dst1d, zblk).reshape(NC, RPAD, F)
    mu, ls = _out_call(t, v2, degT, Wmu, bmu.reshape(1, O),
                       Wls, bls.reshape(1, O))
    return (mu[:N], ls[:N])


# speedup vs baseline: 2.5342x; 2.5342x over previous
import functools

import jax
import jax.numpy as jnp
from jax import lax
from jax.experimental import pallas as pl
from jax.experimental.pallas import tpu as pltpu
from jax.experimental.pallas import tpu_sc as plsc

N = 10000
E = 320000
F = 128
O = 64

NC, NS = 2, 16
NW = NC * NS
CHUNK = 128
CPW = 80
NCHUNKS = CPW * NW
EPAD = NCHUNKS * CHUNK
EPW = CPW * CHUNK
DUMMY = N
DEGW = 128

RPAD = 10240
RPS = RPAD // NS
BLK = 512
NBLK = RPAD // BLK

_f32 = jnp.float32
_mesh = plsc.VectorSubcoreMesh(core_axis_name="c", subcore_axis_name="s")


@functools.partial(
    pl.kernel,
    out_type=jax.ShapeDtypeStruct((NC * RPAD, DEGW), _f32),
    mesh=_mesh,
    scratch_types=[
        pltpu.VMEM((CPW, CHUNK), jnp.int32),
        pltpu.VMEM((CHUNK, DEGW), _f32),
        pltpu.VMEM_SHARED((RPAD, DEGW), _f32),
    ],
)
def _deg_kernel(dst_hbm, ones_hbm, zrow_hbm, out_hbm, dst_v, ones_v, deg_sh):
    c = lax.axis_index("c")
    s = lax.axis_index("s")
    pltpu.sync_copy(zrow_hbm, deg_sh.at[pl.ds(s * RPS, RPS)])
    pltpu.sync_copy(ones_hbm, ones_v)
    cbase = (c * NS + s) * CPW
    pltpu.sync_copy(dst_hbm.at[pl.ds(cbase, CPW)], dst_v)
    plsc.subcore_barrier()

    @pl.loop(0, CPW)
    def _(j):
        pltpu.sync_copy(ones_v, deg_sh.at[dst_v.at[j]], add=True)

    plsc.subcore_barrier()
    pltpu.sync_copy(deg_sh.at[pl.ds(s * RPS, RPS)],
                    out_hbm.at[pl.ds(c * RPAD + s * RPS, RPS)])


@functools.partial(
    pl.kernel,
    out_type=jax.ShapeDtypeStruct((NC * RPAD, F), _f32),
    mesh=_mesh,
    scratch_types=[
        pltpu.VMEM((CPW, CHUNK), jnp.int32),
        pltpu.VMEM((2, CHUNK), jnp.int32),
        pltpu.VMEM((2, CHUNK, F), _f32),
        pltpu.VMEM_SHARED((RPAD, F), _f32),
        pltpu.SemaphoreType.DMA,
        pltpu.SemaphoreType.DMA,
        pltpu.SemaphoreType.DMA,
        pltpu.SemaphoreType.DMA,
    ],
)
def _agg_kernel(v_hbm, src_hbm, dst_hbm, zblk_hbm, out_hbm,
                src_v, dst_v, rows_v, acc_sh, rs0, rs1, ds0, ds1):
    c = lax.axis_index("c")
    s = lax.axis_index("s")
    pltpu.sync_copy(zblk_hbm, acc_sh.at[pl.ds(s * RPS, RPS)])
    cbase = (c * NS + s) * CPW
    pltpu.sync_copy(src_hbm.at[pl.ds(cbase, CPW)], src_v)
    pltpu.sync_copy(dst_hbm.at[cbase], dst_v.at[0])
    plsc.subcore_barrier()
    rsems = (rs0, rs1)
    dsems = (ds0, ds1)

    pltpu.async_copy(v_hbm.at[src_v.at[0]], rows_v.at[0], rs0)

    @pl.loop(0, CPW, step=2)
    def _(j):
        for b in range(2):
            nxt = j + b + 1

            @pl.when(nxt < CPW)
            def _():
                pltpu.async_copy(dst_hbm.at[cbase + nxt], dst_v.at[1 - b],
                                 dsems[1 - b])
                pltpu.async_copy(v_hbm.at[src_v.at[nxt]], rows_v.at[1 - b],
                                 rsems[1 - b])

            @pl.when(j + b > 0)
            def _():
                pltpu.make_async_copy(dst_hbm.at[cbase], dst_v.at[b],
                                      dsems[b]).wait()

            pltpu.make_async_copy(v_hbm.at[pl.ds(0, CHUNK)], rows_v.at[b],
                                  rs0 if b == 0 else rs1).wait()
            pltpu.sync_copy(rows_v.at[b], acc_sh.at[dst_v.at[b]], add=True)

    plsc.subcore_barrier()
    pltpu.sync_copy(acc_sh.at[pl.ds(s * RPS, RPS)],
                    out_hbm.at[pl.ds(c * RPAD + s * RPS, RPS)])


def _dinv_block(degT_blk):
    deg = degT_blk[:, 0:1] + degT_blk[:, 1:2] + 1.0
    return lax.rsqrt(deg)


def _prep_body(x_ref, w1_ref, degT_ref, v1_ref):
    xw = jnp.dot(x_ref[...], w1_ref[...], preferred_element_type=_f32,
                 precision=lax.Precision.HIGHEST)
    v1_ref[...] = xw * _dinv_block(degT_ref[...])


def _mid_body(s_ref, v1_ref, degT_ref, b1_ref, v2_ref):
    dinv = _dinv_block(degT_ref[...])
    pre = (s_ref[0] + s_ref[1] + v1_ref[...]) * dinv + b1_ref[...]
    v2_ref[...] = jnp.maximum(pre, 0.0) * dinv


def _out_body(t_ref, v2_ref, degT_ref, wmu_ref, bmu_ref, wls_ref, bls_ref,
              mu_ref, ls_ref):
    dinv = _dinv_block(degT_ref[...])
    q = (t_ref[0] + t_ref[1] + v2_ref[...]) * dinv
    mu_ref[...] = jnp.dot(q, wmu_ref[...], preferred_element_type=_f32,
                          precision=lax.Precision.HIGHEST) + bmu_ref[...]
    ls_ref[...] = jnp.dot(q, wls_ref[...], preferred_element_type=_f32,
                          precision=lax.Precision.HIGHEST) + bls_ref[...]


_row = lambda i: (i, 0)
_full2 = pl.BlockSpec((2, RPAD), lambda i: (0, 0))

_prep_call = pl.pallas_call(
    _prep_body,
    grid=(NBLK,),
    in_specs=[pl.BlockSpec((BLK, F), _row),
              pl.BlockSpec((F, F), lambda i: (0, 0)),
              pl.BlockSpec((BLK, 2), _row)],
    out_specs=pl.BlockSpec((BLK, F), _row),
    out_shape=jax.ShapeDtypeStruct((RPAD, F), _f32),
)

_mid_call = pl.pallas_call(
    _mid_body,
    grid=(NBLK,),
    in_specs=[pl.BlockSpec((2, BLK, F), lambda i: (0, i, 0)),
              pl.BlockSpec((BLK, F), _row),
              pl.BlockSpec((BLK, 2), _row),
              pl.BlockSpec((1, F), lambda i: (0, 0))],
    out_specs=pl.BlockSpec((BLK, F), _row),
    out_shape=jax.ShapeDtypeStruct((RPAD, F), _f32),
)

_out_call = pl.pallas_call(
    _out_body,
    grid=(NBLK,),
    in_specs=[pl.BlockSpec((2, BLK, F), lambda i: (0, i, 0)),
              pl.BlockSpec((BLK, F), _row),
              pl.BlockSpec((BLK, 2), _row),
              pl.BlockSpec((F, O), lambda i: (0, 0)),
              pl.BlockSpec((1, O), lambda i: (0, 0)),
              pl.BlockSpec((F, O), lambda i: (0, 0)),
              pl.BlockSpec((1, O), lambda i: (0, 0))],
    out_specs=[pl.BlockSpec((BLK, O), _row), pl.BlockSpec((BLK, O), _row)],
    out_shape=[jax.ShapeDtypeStruct((RPAD, O), _f32),
               jax.ShapeDtypeStruct((RPAD, O), _f32)],
)


def kernel(x, edge_index, W1, b1, Wmu, bmu, Wls, bls):
    src = edge_index[0].astype(jnp.int32)
    dst = edge_index[1].astype(jnp.int32)
    pad = EPAD - E
    fill = (jnp.arange(pad, dtype=jnp.int32) % (RPAD - N)) + DUMMY
    src1d = jnp.concatenate([src, fill]).reshape(NCHUNKS, CHUNK)
    dst1d = jnp.concatenate([dst, fill]).reshape(NCHUNKS, CHUNK)

    ones = jnp.ones((CHUNK, DEGW), _f32)
    zrow = jnp.zeros((RPS, DEGW), _f32)
    zblk = jnp.zeros((RPS, F), _f32)

    degp = _deg_kernel(dst1d, ones, zrow)
    degT = degp.reshape(NC, RPAD, DEGW)[:, :, 0].T

    x_pad = jnp.concatenate([x, jnp.zeros((RPAD - N, F), _f32)])
    v1 = _prep_call(x_pad, W1, degT)
    s = _agg_kernel(v1, src1d, dst1d, zblk).reshape(NC, RPAD, F)
    v2 = _mid_call(s, v1, degT, b1.reshape(1, F))
    t = _agg_kernel(v2, src1d, dst1d, zblk).reshape(NC, RPAD, F)
    mu, ls = _out_call(t, v2, degT, Wmu, bmu.reshape(1, O),
                       Wls, bls.reshape(1, O))
    return (mu[:N], ls[:N])


# speedup vs baseline: 3.4408x; 1.3578x over previous
import functools

import jax
import jax.numpy as jnp
from jax import lax
from jax.experimental import pallas as pl
from jax.experimental.pallas import tpu as pltpu
from jax.experimental.pallas import tpu_sc as plsc

N = 10000
E = 320000
F = 128
O = 64

NC, NS = 2, 16
NW = NC * NS
CHUNK = 128
CPW = 80
NCHUNKS = CPW * NW
EPAD = NCHUNKS * CHUNK
EPW = CPW * CHUNK
DUMMY = N

RPAD = 10240
RPS = RPAD // NS
BLK = 512
NBLK = RPAD // BLK

_f32 = jnp.float32
_mesh = plsc.VectorSubcoreMesh(core_axis_name="c", subcore_axis_name="s")


@functools.partial(
    pl.kernel,
    out_type=jax.ShapeDtypeStruct((NW * RPAD,), _f32),
    mesh=_mesh,
    scratch_types=[
        pltpu.VMEM((EPW,), jnp.int32),
        pltpu.VMEM((RPAD,), _f32),
    ],
    compiler_params=pltpu.CompilerParams(needs_layout_passes=False),
)
def _deg_kernel(dst_hbm, zdeg_hbm, out_hbm, dst_v, hist_v):
    c = lax.axis_index("c")
    s = lax.axis_index("s")
    w = c * NS + s
    pltpu.sync_copy(zdeg_hbm, hist_v)
    pltpu.sync_copy(dst_hbm.at[pl.ds(w * EPW, EPW)], dst_v)
    ones16 = jnp.ones((16,), _f32)

    mask16 = jnp.ones((16,), jnp.bool_)

    @pl.loop(0, EPW // 16)
    def _(g):
        idx = dst_v[pl.ds(g * 16, 16)]
        plsc.addupdate_scatter(hist_v, [idx], ones16, mask=mask16)

    pltpu.sync_copy(hist_v, out_hbm.at[pl.ds(w * RPAD, RPAD)])


@functools.partial(
    pl.kernel,
    out_type=jax.ShapeDtypeStruct((NC * RPAD, F), _f32),
    mesh=_mesh,
    scratch_types=[
        pltpu.VMEM((CPW, CHUNK), jnp.int32),
        pltpu.VMEM((2, CHUNK), jnp.int32),
        pltpu.VMEM((2, CHUNK, F), _f32),
        pltpu.VMEM_SHARED((RPAD, F), _f32),
        pltpu.SemaphoreType.DMA,
        pltpu.SemaphoreType.DMA,
        pltpu.SemaphoreType.DMA,
        pltpu.SemaphoreType.DMA,
    ],
)
def _agg_kernel(v_hbm, src_hbm, dst_hbm, zblk_hbm, out_hbm,
                src_v, dst_v, rows_v, acc_sh, rs0, rs1, ds0, ds1):
    c = lax.axis_index("c")
    s = lax.axis_index("s")
    pltpu.sync_copy(zblk_hbm, acc_sh.at[pl.ds(s * RPS, RPS)])
    cbase = (c * NS + s) * CPW
    pltpu.sync_copy(src_hbm.at[pl.ds(cbase, CPW)], src_v)
    pltpu.sync_copy(dst_hbm.at[cbase], dst_v.at[0])
    plsc.subcore_barrier()
    rsems = (rs0, rs1)
    dsems = (ds0, ds1)

    pltpu.async_copy(v_hbm.at[src_v.at[0]], rows_v.at[0], rs0)

    @pl.loop(0, CPW, step=2)
    def _(j):
        for b in range(2):
            nxt = j + b + 1

            @pl.when(nxt < CPW)
            def _():
                pltpu.async_copy(dst_hbm.at[cbase + nxt], dst_v.at[1 - b],
                                 dsems[1 - b])
                pltpu.async_copy(v_hbm.at[src_v.at[nxt]], rows_v.at[1 - b],
                                 rsems[1 - b])

            @pl.when(j + b > 0)
            def _():
                pltpu.make_async_copy(dst_hbm.at[cbase], dst_v.at[b],
                                      dsems[b]).wait()

            pltpu.make_async_copy(v_hbm.at[pl.ds(0, CHUNK)], rows_v.at[b],
                                  rs0 if b == 0 else rs1).wait()
            pltpu.sync_copy(rows_v.at[b], acc_sh.at[dst_v.at[b]], add=True)

    plsc.subcore_barrier()
    pltpu.sync_copy(acc_sh.at[pl.ds(s * RPS, RPS)],
                    out_hbm.at[pl.ds(c * RPAD + s * RPS, RPS)])


def _dinv_block(degT_blk):
    deg = jnp.sum(degT_blk, axis=1, keepdims=True) + 1.0
    return lax.rsqrt(deg)


def _prep_body(x_ref, w1_ref, degT_ref, v1_ref):
    xw = jnp.dot(x_ref[...], w1_ref[...], preferred_element_type=_f32,
                 precision=lax.Precision.HIGHEST)
    v1_ref[...] = xw * _dinv_block(degT_ref[...])


def _mid_body(s_ref, v1_ref, degT_ref, b1_ref, v2_ref):
    dinv = _dinv_block(degT_ref[...])
    pre = (s_ref[0] + s_ref[1] + v1_ref[...]) * dinv + b1_ref[...]
    v2_ref[...] = jnp.maximum(pre, 0.0) * dinv


def _out_body(t_ref, v2_ref, degT_ref, wmu_ref, bmu_ref, wls_ref, bls_ref,
              mu_ref, ls_ref):
    dinv = _dinv_block(degT_ref[...])
    q = (t_ref[0] + t_ref[1] + v2_ref[...]) * dinv
    mu_ref[...] = jnp.dot(q, wmu_ref[...], preferred_element_type=_f32,
                          precision=lax.Precision.HIGHEST) + bmu_ref[...]
    ls_ref[...] = jnp.dot(q, wls_ref[...], preferred_element_type=_f32,
                          precision=lax.Precision.HIGHEST) + bls_ref[...]


_row = lambda i: (i, 0)

_prep_call = pl.pallas_call(
    _prep_body,
    grid=(NBLK,),
    in_specs=[pl.BlockSpec((BLK, F), _row),
              pl.BlockSpec((F, F), lambda i: (0, 0)),
              pl.BlockSpec((BLK, NW), _row)],
    out_specs=pl.BlockSpec((BLK, F), _row),
    out_shape=jax.ShapeDtypeStruct((RPAD, F), _f32),
)

_mid_call = pl.pallas_call(
    _mid_body,
    grid=(NBLK,),
    in_specs=[pl.BlockSpec((2, BLK, F), lambda i: (0, i, 0)),
              pl.BlockSpec((BLK, F), _row),
              pl.BlockSpec((BLK, NW), _row),
              pl.BlockSpec((1, F), lambda i: (0, 0))],
    out_specs=pl.BlockSpec((BLK, F), _row),
    out_shape=jax.ShapeDtypeStruct((RPAD, F), _f32),
)

_out_call = pl.pallas_call(
    _out_body,
    grid=(NBLK,),
    in_specs=[pl.BlockSpec((2, BLK, F), lambda i: (0, i, 0)),
              pl.BlockSpec((BLK, F), _row),
              pl.BlockSpec((BLK, NW), _row),
              pl.BlockSpec((F, O), lambda i: (0, 0)),
              pl.BlockSpec((1, O), lambda i: (0, 0)),
              pl.BlockSpec((F, O), lambda i: (0, 0)),
              pl.BlockSpec((1, O), lambda i: (0, 0))],
    out_specs=[pl.BlockSpec((BLK, O), _row), pl.BlockSpec((BLK, O), _row)],
    out_shape=[jax.ShapeDtypeStruct((RPAD, O), _f32),
               jax.ShapeDtypeStruct((RPAD, O), _f32)],
)


def kernel(x, edge_index, W1, b1, Wmu, bmu, Wls, bls):
    src = edge_index[0].astype(jnp.int32)
    dst = edge_index[1].astype(jnp.int32)
    pad = EPAD - E
    fill = (jnp.arange(pad, dtype=jnp.int32) % (RPAD - N)) + DUMMY
    src_flat = jnp.concatenate([src, fill])
    dst_flat = jnp.concatenate([dst, fill])
    src1d = src_flat.reshape(NCHUNKS, CHUNK)
    dst1d = dst_flat.reshape(NCHUNKS, CHUNK)

    zdeg = jnp.zeros((RPAD,), _f32)
    zblk = jnp.zeros((RPS, F), _f32)

    degp = _deg_kernel(dst_flat, zdeg)
    degT = degp.reshape(NW, RPAD).T

    x_pad = jnp.concatenate([x, jnp.zeros((RPAD - N, F), _f32)])
    v1 = _prep_call(x_pad, W1, degT)
    s = _agg_kernel(v1, src1d, dst1d, zblk).reshape(NC, RPAD, F)
    v2 = _mid_call(s, v1, degT, b1.reshape(1, F))
    t = _agg_kernel(v2, src1d, dst1d, zblk).reshape(NC, RPAD, F)
    mu, ls = _out_call(t, v2, degT, Wmu, bmu.reshape(1, O),
                       Wls, bls.reshape(1, O))
    return (mu[:N], ls[:N])
